# baseline (speedup 1.0000x reference)
import functools

import jax
import jax.numpy as jnp
from jax import lax
from jax.experimental import pallas as pl
from jax.experimental.pallas import tpu as pltpu
from jax.experimental.pallas import tpu_sc as plsc

NC = 2
NS = 16
NW = NC * NS

R = 64
HALF = 25000
W = 2 * HALF
N = 800000
ROWS_PER_W = R // NW

CHUNK = 4000
N_CHUNKS = N // CHUNK
STEPS = CHUNK // 16


def _body(b_hbm, nb_hbm, idx_hbm, out_hbm, table_v, idx_v, out_v):
    wid = lax.axis_index("s") * NC + lax.axis_index("c")
    r0 = wid * ROWS_PER_W

    pltpu.sync_copy(b_hbm.at[pl.ds(r0 * HALF, HALF)], table_v.at[pl.ds(0, HALF)])
    pltpu.sync_copy(nb_hbm.at[pl.ds(r0 * HALF, HALF)], table_v.at[pl.ds(HALF, HALF)])
    pltpu.sync_copy(b_hbm.at[pl.ds((r0 + 1) * HALF, HALF)], table_v.at[pl.ds(W, HALF)])
    pltpu.sync_copy(nb_hbm.at[pl.ds((r0 + 1) * HALF, HALF)], table_v.at[pl.ds(W + HALF, HALF)])

    def chunk_body(c, _):
        base = c * CHUNK
        pltpu.sync_copy(idx_hbm.at[pl.ds(base, CHUNK)], idx_v)

        def gather_body(i, _):
            off = i * 16
            iv = idx_v[pl.ds(off, 16)]
            out_v[pl.ds(off, 16)] = plsc.load_gather(table_v, [iv])
            out_v[pl.ds(CHUNK + off, 16)] = plsc.load_gather(table_v, [iv + W])
            return 0

        lax.fori_loop(0, STEPS, gather_body, 0)
        pltpu.sync_copy(out_v.at[pl.ds(0, CHUNK)],
                        out_hbm.at[pl.ds(r0 * N + base, CHUNK)])
        pltpu.sync_copy(out_v.at[pl.ds(CHUNK, CHUNK)],
                        out_hbm.at[pl.ds((r0 + 1) * N + base, CHUNK)])
        return 0

    lax.fori_loop(0, N_CHUNKS, chunk_body, 0)


@jax.jit
def _run(b_Q, nb_Q, idx):
    mesh = plsc.VectorSubcoreMesh(core_axis_name="c", subcore_axis_name="s")
    flat = pl.kernel(
        _body,
        out_type=jax.ShapeDtypeStruct((R * N,), jnp.float32),
        mesh=mesh,
        scratch_types=[
            pltpu.VMEM((2 * W,), jnp.float32),
            pltpu.VMEM((CHUNK,), jnp.int32),
            pltpu.VMEM((ROWS_PER_W * CHUNK,), jnp.float32),
        ],
        compiler_params=pltpu.CompilerParams(needs_layout_passes=False),
    )(b_Q.reshape(-1), nb_Q.reshape(-1), idx)
    return flat.reshape(R, N)


def kernel(b_Q, nb_Q, all_indices):
    return _run(b_Q, nb_Q, all_indices.astype(jnp.int32))


# speedup vs baseline: 1.0975x; 1.0975x over previous
import jax
import jax.numpy as jnp
from jax import lax
from jax.experimental import pallas as pl
from jax.experimental.pallas import tpu as pltpu
from jax.experimental.pallas import tpu_sc as plsc

NC = 2
NS = 16
NW = NC * NS

R = 64
HALF = 25000
W = 2 * HALF
N = 800000
ROWS_PER_W = R // NW

CHUNK = 4000
N_CHUNKS = N // CHUNK
STEPS = CHUNK // 16
NBUF = 2


def _body(b_hbm, nb_hbm, idx_hbm, out_hbm, table_v, idx_v, out_v,
          sin0, sin1, sout0, sout1):
    wid = lax.axis_index("s") * NC + lax.axis_index("c")
    r0 = wid * ROWS_PER_W
    sin = [sin0, sin1]
    sout = [sout0, sout1]

    pltpu.sync_copy(b_hbm.at[pl.ds(r0 * HALF, HALF)], table_v.at[pl.ds(0, HALF)])
    pltpu.sync_copy(nb_hbm.at[pl.ds(r0 * HALF, HALF)], table_v.at[pl.ds(HALF, HALF)])
    pltpu.sync_copy(b_hbm.at[pl.ds((r0 + 1) * HALF, HALF)], table_v.at[pl.ds(W, HALF)])
    pltpu.sync_copy(nb_hbm.at[pl.ds((r0 + 1) * HALF, HALF)], table_v.at[pl.ds(W + HALF, HALF)])

    def idx_start(cc, b):
        pltpu.async_copy(idx_hbm.at[pl.ds(cc * CHUNK, CHUNK)],
                         idx_v.at[pl.ds(b * CHUNK, CHUNK)], sin[b])

    def out_copies(cc, b):
        base = cc * CHUNK
        c0 = pltpu.make_async_copy(out_v.at[pl.ds(2 * b * CHUNK, CHUNK)],
                                   out_hbm.at[pl.ds(r0 * N + base, CHUNK)], sout[b])
        c1 = pltpu.make_async_copy(out_v.at[pl.ds((2 * b + 1) * CHUNK, CHUNK)],
                                   out_hbm.at[pl.ds((r0 + 1) * N + base, CHUNK)], sout[b])
        return c0, c1

    idx_start(0, 0)
    idx_start(1, 1)

    def super_body(c, _):
        for b in range(NBUF):
            cc = NBUF * c + b

            pltpu.make_async_copy(idx_hbm.at[pl.ds(cc * CHUNK, CHUNK)],
                                  idx_v.at[pl.ds(b * CHUNK, CHUNK)], sin[b]).wait()

            @pl.when(cc >= NBUF)
            def _():
                c0, c1 = out_copies(cc - NBUF, b)
                c0.wait()
                c1.wait()

            @plsc.parallel_loop(0, STEPS, unroll=8)
            def gather_body(i):
                off = i * 16
                iv = idx_v[pl.ds(b * CHUNK + off, 16)]
                out_v[pl.ds(2 * b * CHUNK + off, 16)] = plsc.load_gather(table_v, [iv])
                out_v[pl.ds((2 * b + 1) * CHUNK + off, 16)] = plsc.load_gather(table_v, [iv + W])

            c0, c1 = out_copies(cc, b)
            c0.start()
            c1.start()

            @pl.when(cc + NBUF < N_CHUNKS)
            def _():
                idx_start(cc + NBUF, b)
        return 0

    lax.fori_loop(0, N_CHUNKS // NBUF, super_body, 0)

    for b in range(NBUF):
        c0, c1 = out_copies(N_CHUNKS - NBUF + b, b)
        c0.wait()
        c1.wait()


@jax.jit
def _run(b_Q, nb_Q, idx):
    mesh = plsc.VectorSubcoreMesh(core_axis_name="c", subcore_axis_name="s")
    flat = pl.kernel(
        _body,
        out_type=jax.ShapeDtypeStruct((R * N,), jnp.float32),
        mesh=mesh,
        scratch_types=[
            pltpu.VMEM((2 * W,), jnp.float32),
            pltpu.VMEM((NBUF * CHUNK,), jnp.int32),
            pltpu.VMEM((NBUF * 2 * CHUNK,), jnp.float32),
            pltpu.SemaphoreType.DMA,
            pltpu.SemaphoreType.DMA,
            pltpu.SemaphoreType.DMA,
            pltpu.SemaphoreType.DMA,
        ],
        compiler_params=pltpu.CompilerParams(needs_layout_passes=False),
    )(b_Q.reshape(-1), nb_Q.reshape(-1), idx)
    return flat.reshape(R, N)


def kernel(b_Q, nb_Q, all_indices):
    return _run(b_Q, nb_Q, all_indices.astype(jnp.int32))


# speedup vs baseline: 23.6018x; 21.5045x over previous
import jax
import jax.numpy as jnp
from jax import lax
from jax.experimental import pallas as pl
from jax.experimental.pallas import tpu as pltpu
from jax.experimental.pallas import tpu_sc as plsc

NC = 2
NS = 16
NW = NC * NS

R = 64
HALF = 25000
W = 2 * HALF
N = 800000
ROWS_PER_W = R // NW

CHUNK = 3200
N_CHUNKS = N // CHUNK
STEPS = CHUNK // 16
NBUF = 2


def _body(b_hbm, nb_hbm, idx_hbm, out_hbm, table_v, idx_v, out_v,
          sin0, sin1, sout0, sout1):
    wid = lax.axis_index("s") * NC + lax.axis_index("c")
    r0 = wid * ROWS_PER_W
    sin = [sin0, sin1]
    sout = [sout0, sout1]

    pltpu.sync_copy(b_hbm.at[pl.ds(r0 * HALF, HALF)], table_v.at[pl.ds(0, HALF)])
    pltpu.sync_copy(nb_hbm.at[pl.ds(r0 * HALF, HALF)], table_v.at[pl.ds(HALF, HALF)])
    pltpu.sync_copy(b_hbm.at[pl.ds((r0 + 1) * HALF, HALF)], table_v.at[pl.ds(W, HALF)])
    pltpu.sync_copy(nb_hbm.at[pl.ds((r0 + 1) * HALF, HALF)], table_v.at[pl.ds(W + HALF, HALF)])

    def idx_start(cc, b):
        pltpu.async_copy(idx_hbm.at[pl.ds(cc * CHUNK, CHUNK)],
                         idx_v.at[pl.ds(b * CHUNK, CHUNK)], sin[b])

    def out_copies(cc, b):
        base = cc * CHUNK
        c0 = pltpu.make_async_copy(out_v.at[pl.ds(2 * b * CHUNK, CHUNK)],
                                   out_hbm.at[r0, pl.ds(base, CHUNK)], sout[b])
        c1 = pltpu.make_async_copy(out_v.at[pl.ds((2 * b + 1) * CHUNK, CHUNK)],
                                   out_hbm.at[r0 + 1, pl.ds(base, CHUNK)], sout[b])
        return c0, c1

    idx_start(0, 0)
    idx_start(1, 1)

    def super_body(c, _):
        for b in range(NBUF):
            cc = NBUF * c + b

            pltpu.make_async_copy(idx_hbm.at[pl.ds(cc * CHUNK, CHUNK)],
                                  idx_v.at[pl.ds(b * CHUNK, CHUNK)], sin[b]).wait()

            @pl.when(cc >= NBUF)
            def _():
                c0, c1 = out_copies(cc - NBUF, b)
                c0.wait()
                c1.wait()

            @plsc.parallel_loop(0, STEPS, unroll=8)
            def gather_body(i):
                off = i * 16
                iv = idx_v[pl.ds(b * CHUNK + off, 16)]
                out_v[pl.ds(2 * b * CHUNK + off, 16)] = plsc.load_gather(table_v, [iv])
                out_v[pl.ds((2 * b + 1) * CHUNK + off, 16)] = plsc.load_gather(table_v, [iv + W])

            c0, c1 = out_copies(cc, b)
            c0.start()
            c1.start()

            @pl.when(cc + NBUF < N_CHUNKS)
            def _():
                idx_start(cc + NBUF, b)
        return 0

    lax.fori_loop(0, N_CHUNKS // NBUF, super_body, 0)

    for b in range(NBUF):
        c0, c1 = out_copies(N_CHUNKS - NBUF + b, b)
        c0.wait()
        c1.wait()


@jax.jit
def _run(b_Q, nb_Q, idx):
    mesh = plsc.VectorSubcoreMesh(core_axis_name="c", subcore_axis_name="s")
    return pl.kernel(
        _body,
        out_type=jax.ShapeDtypeStruct((R, N), jnp.float32),
        mesh=mesh,
        scratch_types=[
            pltpu.VMEM((2 * W,), jnp.float32),
            pltpu.VMEM((NBUF * CHUNK,), jnp.int32),
            pltpu.VMEM((NBUF * 2 * CHUNK,), jnp.float32),
            pltpu.SemaphoreType.DMA,
            pltpu.SemaphoreType.DMA,
            pltpu.SemaphoreType.DMA,
            pltpu.SemaphoreType.DMA,
        ],
        compiler_params=pltpu.CompilerParams(needs_layout_passes=False),
    )(b_Q.reshape(-1), nb_Q.reshape(-1), idx)


def kernel(b_Q, nb_Q, all_indices):
    return _run(b_Q, nb_Q, all_indices.astype(jnp.int32))


# speedup vs baseline: 35.2637x; 1.4941x over previous
import jax
import jax.numpy as jnp
from jax import lax
from jax.experimental import pallas as pl
from jax.experimental.pallas import tpu as pltpu
from jax.experimental.pallas import tpu_sc as plsc

NC = 2
NS = 16
NW = NC * NS

R = 64
HALF = 25000
W = 2 * HALF
N = 800000
ROWS_PER_W = R // NW

CHUNK = 3200
N_CHUNKS = N // CHUNK
STEPS = CHUNK // 16
NBUF = 2

CH_PER_SUPER = 10
SUPER = CH_PER_SUPER * CHUNK
N_SUPER = N_CHUNKS // CH_PER_SUPER


def _body(b_hbm, nb_hbm, idx_hbm, out_hbm, table_v, idx_v, out_v, idx_s,
          sin0, sin1, sout0, sout1, sload):
    sid = lax.axis_index("s")
    wid = sid * NC + lax.axis_index("c")
    r0 = wid * ROWS_PER_W
    sin = [sin0, sin1]
    sout = [sout0, sout1]

    @pl.when(sid == 0)
    def _():
        pltpu.sync_copy(idx_hbm.at[pl.ds(0, SUPER)], idx_s.at[pl.ds(0, SUPER)])

    pltpu.sync_copy(b_hbm.at[pl.ds(r0 * HALF, HALF)], table_v.at[pl.ds(0, HALF)])
    pltpu.sync_copy(nb_hbm.at[pl.ds(r0 * HALF, HALF)], table_v.at[pl.ds(HALF, HALF)])
    pltpu.sync_copy(b_hbm.at[pl.ds((r0 + 1) * HALF, HALF)], table_v.at[pl.ds(W, HALF)])
    pltpu.sync_copy(nb_hbm.at[pl.ds((r0 + 1) * HALF, HALF)], table_v.at[pl.ds(W + HALF, HALF)])

    plsc.subcore_barrier()

    def idx_start(p, lc, b):
        pltpu.async_copy(idx_s.at[pl.ds(p * SUPER + lc * CHUNK, CHUNK)],
                         idx_v.at[pl.ds(b * CHUNK, CHUNK)], sin[b])

    def idx_wait(p, lc, b):
        pltpu.make_async_copy(idx_s.at[pl.ds(p * SUPER + lc * CHUNK, CHUNK)],
                              idx_v.at[pl.ds(b * CHUNK, CHUNK)], sin[b]).wait()

    def out_copies(cc, b):
        base = cc * CHUNK
        c0 = pltpu.make_async_copy(out_v.at[pl.ds(2 * b * CHUNK, CHUNK)],
                                   out_hbm.at[r0, pl.ds(base, CHUNK)], sout[b])
        c1 = pltpu.make_async_copy(out_v.at[pl.ds((2 * b + 1) * CHUNK, CHUNK)],
                                   out_hbm.at[r0 + 1, pl.ds(base, CHUNK)], sout[b])
        return c0, c1

    def super_loop(s, _):
        p = lax.rem(s, 2)
        pnext = 1 - p

        @pl.when((sid == 0) & (s + 1 < N_SUPER))
        def _():
            pltpu.async_copy(idx_hbm.at[pl.ds((s + 1) * SUPER, SUPER)],
                             idx_s.at[pl.ds(pnext * SUPER, SUPER)], sload)

        idx_start(p, 0, 0)
        idx_start(p, 1, 1)

        def chunk_pair(c, _):
            for b in range(NBUF):
                lc = NBUF * c + b
                cc = s * CH_PER_SUPER + lc

                idx_wait(p, lc, b)

                @pl.when(cc >= NBUF)
                def _():
                    c0, c1 = out_copies(cc - NBUF, b)
                    c0.wait()
                    c1.wait()

                @plsc.parallel_loop(0, STEPS, unroll=8)
                def gather_body(i):
                    off = i * 16
                    iv = idx_v[pl.ds(b * CHUNK + off, 16)]
                    out_v[pl.ds(2 * b * CHUNK + off, 16)] = plsc.load_gather(table_v, [iv])
                    out_v[pl.ds((2 * b + 1) * CHUNK + off, 16)] = plsc.load_gather(table_v, [iv + W])

                c0, c1 = out_copies(cc, b)
                c0.start()
                c1.start()

                @pl.when(lc + NBUF < CH_PER_SUPER)
                def _():
                    idx_start(p, lc + NBUF, b)
            return 0

        lax.fori_loop(0, CH_PER_SUPER // NBUF, chunk_pair, 0)

        @pl.when((sid == 0) & (s + 1 < N_SUPER))
        def _():
            pltpu.make_async_copy(idx_hbm.at[pl.ds((s + 1) * SUPER, SUPER)],
                                  idx_s.at[pl.ds(pnext * SUPER, SUPER)], sload).wait()

        plsc.subcore_barrier()
        return 0

    lax.fori_loop(0, N_SUPER, super_loop, 0)

    for b in range(NBUF):
        c0, c1 = out_copies(N_CHUNKS - NBUF + b, b)
        c0.wait()
        c1.wait()


@jax.jit
def _run(b_Q, nb_Q, idx):
    mesh = plsc.VectorSubcoreMesh(core_axis_name="c", subcore_axis_name="s")
    return pl.kernel(
        _body,
        out_type=jax.ShapeDtypeStruct((R, N), jnp.float32),
        mesh=mesh,
        scratch_types=[
            pltpu.VMEM((2 * W,), jnp.float32),
            pltpu.VMEM((NBUF * CHUNK,), jnp.int32),
            pltpu.VMEM((NBUF * 2 * CHUNK,), jnp.float32),
            pltpu.VMEM_SHARED((2 * SUPER,), jnp.int32),
            pltpu.SemaphoreType.DMA,
            pltpu.SemaphoreType.DMA,
            pltpu.SemaphoreType.DMA,
            pltpu.SemaphoreType.DMA,
            pltpu.SemaphoreType.DMA,
        ],
        compiler_params=pltpu.CompilerParams(needs_layout_passes=False),
    )(b_Q.reshape(-1), nb_Q.reshape(-1), idx)


def kernel(b_Q, nb_Q, all_indices):
    return _run(b_Q, nb_Q, all_indices.astype(jnp.int32))


# speedup vs baseline: 35.6799x; 1.0118x over previous
import jax
import jax.numpy as jnp
from jax import lax
from jax.experimental import pallas as pl
from jax.experimental.pallas import tpu as pltpu
from jax.experimental.pallas import tpu_sc as plsc

NC = 2
NS = 16
NW = NC * NS

R = 64
HALF = 25000
W = 2 * HALF
N = 800000
ROWS_PER_W = R // NW

CHUNK = 3200
N_CHUNKS = N // CHUNK
STEPS = CHUNK // 16
NBUF = 2

CH_PER_SUPER = 10
SUPER = CH_PER_SUPER * CHUNK
N_SUPER = N_CHUNKS // CH_PER_SUPER


def _body(b_hbm, nb_hbm, idx_hbm, out_hbm, table_v, idx_v, out_v, idx_s,
          sin0, sin1, sout0, sout1, sload, stab):
    sid = lax.axis_index("s")
    wid = sid * NC + lax.axis_index("c")
    r0 = wid * ROWS_PER_W
    sin = [sin0, sin1]
    sout = [sout0, sout1]

    @pl.when(sid == 0)
    def _():
        pltpu.sync_copy(idx_hbm.at[pl.ds(0, SUPER)], idx_s.at[pl.ds(0, SUPER)])

    tcopies = [
        pltpu.make_async_copy(b_hbm.at[pl.ds(r0 * HALF, HALF)],
                              table_v.at[pl.ds(0, HALF)], stab),
        pltpu.make_async_copy(nb_hbm.at[pl.ds(r0 * HALF, HALF)],
                              table_v.at[pl.ds(HALF, HALF)], stab),
        pltpu.make_async_copy(b_hbm.at[pl.ds((r0 + 1) * HALF, HALF)],
                              table_v.at[pl.ds(W, HALF)], stab),
        pltpu.make_async_copy(nb_hbm.at[pl.ds((r0 + 1) * HALF, HALF)],
                              table_v.at[pl.ds(W + HALF, HALF)], stab),
    ]
    for c in tcopies:
        c.start()
    for c in tcopies:
        c.wait()

    plsc.subcore_barrier()

    def idx_start(p, lc, b):
        pltpu.async_copy(idx_s.at[pl.ds(p * SUPER + lc * CHUNK, CHUNK)],
                         idx_v.at[pl.ds(b * CHUNK, CHUNK)], sin[b])

    def idx_wait(p, lc, b):
        pltpu.make_async_copy(idx_s.at[pl.ds(p * SUPER + lc * CHUNK, CHUNK)],
                              idx_v.at[pl.ds(b * CHUNK, CHUNK)], sin[b]).wait()

    def out_copies(cc, b):
        base = cc * CHUNK
        c0 = pltpu.make_async_copy(out_v.at[pl.ds(2 * b * CHUNK, CHUNK)],
                                   out_hbm.at[r0, pl.ds(base, CHUNK)], sout[b])
        c1 = pltpu.make_async_copy(out_v.at[pl.ds((2 * b + 1) * CHUNK, CHUNK)],
                                   out_hbm.at[r0 + 1, pl.ds(base, CHUNK)], sout[b])
        return c0, c1

    def super_loop(s, _):
        p = lax.rem(s, 2)
        pnext = 1 - p

        @pl.when((sid == 0) & (s + 1 < N_SUPER))
        def _():
            pltpu.async_copy(idx_hbm.at[pl.ds((s + 1) * SUPER, SUPER)],
                             idx_s.at[pl.ds(pnext * SUPER, SUPER)], sload)

        idx_start(p, 0, 0)
        idx_start(p, 1, 1)

        def chunk_pair(c, _):
            for b in range(NBUF):
                lc = NBUF * c + b
                cc = s * CH_PER_SUPER + lc

                idx_wait(p, lc, b)

                @pl.when(cc >= NBUF)
                def _():
                    c0, c1 = out_copies(cc - NBUF, b)
                    c0.wait()
                    c1.wait()

                @plsc.parallel_loop(0, STEPS, unroll=8)
                def gather_body(i):
                    off = i * 16
                    iv = idx_v[pl.ds(b * CHUNK + off, 16)]
                    out_v[pl.ds(2 * b * CHUNK + off, 16)] = plsc.load_gather(table_v, [iv])
                    out_v[pl.ds((2 * b + 1) * CHUNK + off, 16)] = plsc.load_gather(table_v, [iv + W])

                c0, c1 = out_copies(cc, b)
                c0.start()
                c1.start()

                @pl.when(lc + NBUF < CH_PER_SUPER)
                def _():
                    idx_start(p, lc + NBUF, b)
            return 0

        lax.fori_loop(0, CH_PER_SUPER // NBUF, chunk_pair, 0)

        @pl.when((sid == 0) & (s + 1 < N_SUPER))
        def _():
            pltpu.make_async_copy(idx_hbm.at[pl.ds((s + 1) * SUPER, SUPER)],
                                  idx_s.at[pl.ds(pnext * SUPER, SUPER)], sload).wait()

        plsc.subcore_barrier()
        return 0

    lax.fori_loop(0, N_SUPER, super_loop, 0)

    for b in range(NBUF):
        c0, c1 = out_copies(N_CHUNKS - NBUF + b, b)
        c0.wait()
        c1.wait()


@jax.jit
def _run(b_Q, nb_Q, idx):
    mesh = plsc.VectorSubcoreMesh(core_axis_name="c", subcore_axis_name="s")
    return pl.kernel(
        _body,
        out_type=jax.ShapeDtypeStruct((R, N), jnp.float32),
        mesh=mesh,
        scratch_types=[
            pltpu.VMEM((2 * W,), jnp.float32),
            pltpu.VMEM((NBUF * CHUNK,), jnp.int32),
            pltpu.VMEM((NBUF * 2 * CHUNK,), jnp.float32),
            pltpu.VMEM_SHARED((2 * SUPER,), jnp.int32),
            pltpu.SemaphoreType.DMA,
            pltpu.SemaphoreType.DMA,
            pltpu.SemaphoreType.DMA,
            pltpu.SemaphoreType.DMA,
            pltpu.SemaphoreType.DMA,
            pltpu.SemaphoreType.DMA,
        ],
        compiler_params=pltpu.CompilerParams(needs_layout_passes=False),
    )(b_Q.reshape(-1), nb_Q.reshape(-1), idx)


def kernel(b_Q, nb_Q, all_indices):
    return _run(b_Q, nb_Q, all_indices.astype(jnp.int32))
